# baseline (speedup 1.0000x reference)
import jax
import jax.numpy as jnp
from jax.experimental import pallas as pl
from jax.experimental.pallas import tpu as pltpu

_SUB = 16
_LANES = 128
_CHUNK = _SUB * _LANES
_NW = 151


def _round_up(n, m):
    return ((n + m - 1) // m) * m


def _tree_sum(terms):
    while len(terms) > 1:
        nxt = [terms[i] + terms[i + 1] for i in range(0, len(terms) - 1, 2)]
        if len(terms) % 2:
            nxt.append(terms[-1])
        terms = nxt
    return terms[0]


def _mlp_chunk(x_ref, w_scr, o_ref, s):
    x0 = x_ref[0, pl.ds(s, _SUB), :]
    x1 = x_ref[1, pl.ds(s, _SUB), :]

    w = [w_scr[j] for j in range(_NW)]
    w1 = w[0:20]
    b1 = w[20:30]
    w2 = w[30:130]
    b2 = w[130:140]
    w3 = w[140:150]
    b3 = w[150]

    h1 = [jnp.maximum(w1[2 * j] * x0 + (w1[2 * j + 1] * x1 + b1[j]),
                      jnp.bfloat16(0))
          for j in range(10)]

    h2 = []
    for j in range(10):
        prods = [w2[j * 10 + k] * h1[k] for k in range(10)]
        prods.append(b2[j])
        h2.append(jnp.maximum(_tree_sum(prods), jnp.bfloat16(0)))

    prods = [w3[k] * h2[k] for k in range(10)]
    prods.append(b3)

    z = _tree_sum(prods).astype(jnp.float32)
    o_ref[pl.ds(s, _SUB), :] = 0.5 * (jnp.tanh(0.5 * z) + 1.0)


def _mlp_kernel(x_ref, wf_ref, o_ref, w_scr):
    @pl.when(pl.program_id(0) == 0)
    def _fill():
        for j in range(_NW):
            w_scr[j] = jnp.full((_SUB, _LANES), wf_ref[j], jnp.bfloat16)

    n = o_ref.shape[0] // _SUB
    for c in range(n):
        _mlp_chunk(x_ref, w_scr, o_ref, c * _SUB)


def kernel(x, w1, b1, w2, b2, w3, b3):
    B = x.shape[0]
    tile_b = min(65536, _round_up(pl.cdiv(B, 8), _CHUNK))
    tile_b = max(_CHUNK, _round_up(tile_b, _CHUNK))
    Bp = _round_up(B, tile_b)
    n_tiles = Bp // tile_b
    c_tile = tile_b // _LANES

    xt = (jnp.pad(x.T, ((0, 0), (0, Bp - B)))
          .astype(jnp.bfloat16)
          .reshape(2, Bp // _LANES, _LANES))

    wf = jnp.concatenate([
        w1.reshape(-1), b1.reshape(-1),
        w2.reshape(-1), b2.reshape(-1),
        w3.reshape(-1), b3.reshape(-1),
    ]).astype(jnp.float32)

    out = pl.pallas_call(
        _mlp_kernel,
        out_shape=jax.ShapeDtypeStruct((Bp // _LANES, _LANES), jnp.float32),
        grid=(n_tiles,),
        in_specs=[
            pl.BlockSpec((2, c_tile, _LANES), lambda i: (0, i, 0)),
            pl.BlockSpec(memory_space=pltpu.SMEM),
        ],
        out_specs=pl.BlockSpec((c_tile, _LANES), lambda i: (i, 0)),
        scratch_shapes=[pltpu.VMEM((_NW, _SUB, _LANES), jnp.bfloat16)],
        compiler_params=pltpu.CompilerParams(
            dimension_semantics=("parallel",),
        ),
    )(xt, wf)

    return out.reshape(Bp)[:B].reshape(B, 1)


# speedup vs baseline: 1.3307x; 1.3307x over previous
import jax
import jax.numpy as jnp
from jax.experimental import pallas as pl
from jax.experimental.pallas import tpu as pltpu

_SUB = 8
_LANES = 256
_CHUNK = _SUB * _LANES


def _round_up(n, m):
    return ((n + m - 1) // m) * m


def _tree_sum(terms):
    while len(terms) > 1:
        nxt = [terms[i] + terms[i + 1] for i in range(0, len(terms) - 1, 2)]
        if len(terms) % 2:
            nxt.append(terms[-1])
        terms = nxt
    return terms[0]


def _mlp_chunk(x_ref, w1_ref, b1_ref, w2_ref, b2_ref, w3_ref, b3_ref, o_ref, s):
    x0 = x_ref[0, pl.ds(s, _SUB), :]
    x1 = x_ref[1, pl.ds(s, _SUB), :]

    h1 = [jnp.maximum(w1_ref[2 * j] * x0 + (w1_ref[2 * j + 1] * x1 + b1_ref[j]),
                      jnp.bfloat16(0))
          for j in range(10)]

    h2 = []
    for j in range(10):
        prods = [w2_ref[j * 10 + k] * h1[k] for k in range(10)]
        prods.append(b2_ref[j])
        h2.append(jnp.maximum(_tree_sum(prods), jnp.bfloat16(0)))

    prods = [w3_ref[k] * h2[k] for k in range(10)]
    prods.append(b3_ref[0])

    z = _tree_sum(prods).astype(jnp.float32)
    o_ref[pl.ds(s, _SUB), :] = 0.5 * (jnp.tanh(0.5 * z) + 1.0)


def _mlp_kernel(x_ref, w1_ref, b1_ref, w2_ref, b2_ref, w3_ref, b3_ref, o_ref):
    n = x_ref.shape[1] // _SUB
    refs = (x_ref, w1_ref, b1_ref, w2_ref, b2_ref, w3_ref, b3_ref, o_ref)
    for c in range(n):
        _mlp_chunk(*refs, c * _SUB)


def kernel(x, w1, b1, w2, b2, w3, b3):
    B = x.shape[0]
    tile_b = min(131072, _round_up(pl.cdiv(B, 8), _CHUNK))
    tile_b = max(_CHUNK, _round_up(tile_b, _CHUNK))
    Bp = _round_up(B, tile_b)
    n_tiles = Bp // tile_b
    c_tile = tile_b // _LANES

    xt = (jnp.pad(x.T, ((0, 0), (0, Bp - B)))
          .astype(jnp.bfloat16)
          .reshape(2, Bp // _LANES, _LANES))

    def splat(a):
        flat = a.reshape(-1).astype(jnp.bfloat16)
        return jnp.broadcast_to(flat[:, None, None],
                                (flat.shape[0], _SUB, _LANES))

    w1b, b1b = splat(w1), splat(b1)
    w2b, b2b = splat(w2), splat(b2)
    w3b, b3b = splat(w3), splat(b3)

    def const_spec(nrows):
        return pl.BlockSpec((nrows, _SUB, _LANES), lambda i: (0, 0, 0))

    out = pl.pallas_call(
        _mlp_kernel,
        out_shape=jax.ShapeDtypeStruct((Bp // _LANES, _LANES), jnp.float32),
        grid=(n_tiles,),
        in_specs=[
            pl.BlockSpec((2, c_tile, _LANES), lambda i: (0, i, 0)),
            const_spec(20), const_spec(10),
            const_spec(100), const_spec(10),
            const_spec(10), const_spec(1),
        ],
        out_specs=pl.BlockSpec((c_tile, _LANES), lambda i: (i, 0)),
        compiler_params=pltpu.CompilerParams(
            dimension_semantics=("parallel",),
        ),
    )(xt, w1b, b1b, w2b, b2b, w3b, b3b)

    return out.reshape(Bp)[:B].reshape(B, 1)
